# baseline (speedup 1.0000x reference)
import functools

import jax
import jax.numpy as jnp
from jax import lax
from jax.experimental import pallas as pl
from jax.experimental.pallas import tpu as pltpu
from jax.experimental.pallas import tpu_sc as plsc

E = 160000
N = 10000
D = 256
DH = D // 2
NC, NS, L = 2, 16, 16
NW = NC * NS

BM = 2000


def _mm_body(m_ref, wl_ref, bl_ref, wr_ref, br_ref, hl_ref, hr_ref):
    x = m_ref[...]
    dn = (((1,), (1,)), ((), ()))
    hl_ref[...] = lax.dot_general(
        x, wl_ref[...], dn, preferred_element_type=jnp.float32,
        precision=lax.Precision.HIGHEST) + bl_ref[...]
    hr_ref[...] = lax.dot_general(
        x, wr_ref[...], dn, preferred_element_type=jnp.float32,
        precision=lax.Precision.HIGHEST) + br_ref[...]


def _linear_parts(M, Wl, bl, Wr, br):
    return pl.pallas_call(
        _mm_body,
        grid=(E // BM,),
        in_specs=[
            pl.BlockSpec((BM, D), lambda i: (i, 0)),
            pl.BlockSpec((D, D), lambda i: (0, 0)),
            pl.BlockSpec((1, D), lambda i: (0, 0)),
            pl.BlockSpec((D, D), lambda i: (0, 0)),
            pl.BlockSpec((1, D), lambda i: (0, 0)),
        ],
        out_specs=[pl.BlockSpec((BM, D), lambda i: (i, 0)),
                   pl.BlockSpec((BM, D), lambda i: (i, 0))],
        out_shape=[jax.ShapeDtypeStruct((E, D), jnp.float32),
                   jax.ShapeDtypeStruct((E, D), jnp.float32)],
    )(M, Wl, bl.reshape(1, D), Wr, br.reshape(1, D))


_SC_MESH = plsc.VectorSubcoreMesh(core_axis_name="c", subcore_axis_name="s")

_GDN = lax.GatherDimensionNumbers(
    offset_dims=(), collapsed_slice_dims=(0,), start_index_map=(0,))


def _shuffle(v, idx16):
    return lax.gather(v, idx16.reshape(L, 1), _GDN, (1,),
                      mode=lax.GatherScatterMode.PROMISE_IN_BOUNDS)


def _hsum(v):
    lanes = lax.iota(jnp.int32, L)
    for sh in (1, 2, 4, 8):
        v = v + _shuffle(v, lanes ^ sh)
    return v


EPT2 = E // NW
B2 = 40
NB2 = EPT2 // B2


def _scores_call(Hl, Hr, rev, wa):
    @functools.partial(
        pl.kernel,
        out_type=jax.ShapeDtypeStruct((E,), jnp.float32),
        mesh=_SC_MESH,
        compiler_params=pltpu.CompilerParams(needs_layout_passes=False),
        scratch_types=[
            pltpu.VMEM((D,), jnp.float32),
            pltpu.VMEM((3, B2), jnp.int32),
            pltpu.VMEM((3, B2, D), jnp.float32),
            pltpu.VMEM((3, B2, D), jnp.float32),
            pltpu.VMEM((3, B2), jnp.float32),
            pltpu.SemaphoreType.DMA,
            pltpu.SemaphoreType.DMA,
            pltpu.SemaphoreType.DMA,
            pltpu.SemaphoreType.DMA,
            pltpu.SemaphoreType.DMA,
            pltpu.SemaphoreType.DMA,
            pltpu.SemaphoreType.DMA,
            pltpu.SemaphoreType.DMA,
            pltpu.SemaphoreType.DMA,
            pltpu.SemaphoreType.DMA,
            pltpu.SemaphoreType.DMA,
            pltpu.SemaphoreType.DMA,
        ],
    )
    def k(hl_hbm, hr_hbm, rev_hbm, wa_hbm, out_hbm,
          wa_v, rev_v, hl_v, hr_v, s_v,
          semR0, semR1, semR2, semH0, semH1, semH2,
          semG0, semG1, semG2, semO0, semO1, semO2):
        semR = (semR0, semR1, semR2)
        semH = (semH0, semH1, semH2)
        semG = (semG0, semG1, semG2)
        semO = (semO0, semO1, semO2)
        wid = lax.axis_index("s") * NC + lax.axis_index("c")
        base0 = wid * EPT2
        pltpu.sync_copy(wa_hbm, wa_v)
        lane0 = lax.iota(jnp.int32, L) == 0

        def fire_a(j, b):
            base = base0 + j * B2
            @pl.when(j >= 3)
            def _():
                pltpu.make_async_copy(
                    s_v.at[b], out_hbm.at[pl.ds(base0 + (j - 3) * B2, B2)],
                    semO[b]).wait()

            pltpu.async_copy(rev_hbm.at[pl.ds(base, B2)], rev_v.at[b], semR[b])
            pltpu.async_copy(hl_hbm.at[pl.ds(base, B2)], hl_v.at[b], semH[b])

        def fire_b(j, b):
            base = base0 + j * B2
            pltpu.make_async_copy(rev_hbm.at[pl.ds(base, B2)], rev_v.at[b],
                                  semR[b]).wait()
            pltpu.async_copy(hr_hbm.at[rev_v.at[b]], hr_v.at[b], semG[b])

        def work(j, b):
            base = base0 + j * B2
            pltpu.make_async_copy(hl_hbm.at[pl.ds(base, B2)], hl_v.at[b],
                                  semH[b]).wait()
            pltpu.make_async_copy(hr_hbm.at[rev_v.at[b]], hr_v.at[b],
                                  semG[b]).wait()

            def edge(i, c2):
                acc = jnp.zeros((L,), jnp.float32)
                for k16 in range(D // L):
                    sl = pl.ds(k16 * L, L)
                    t = hl_v[b, i, sl] + hr_v[b, i, sl]
                    t = jnp.where(t >= 0.0, t, 0.2 * t)
                    acc = acc + t * wa_v[sl]
                plsc.store_scatter(s_v.at[b], [jnp.full((L,), i, jnp.int32)],
                                   _hsum(acc), mask=lane0)
                return c2

            lax.fori_loop(0, B2, edge, 0)
            pltpu.async_copy(s_v.at[b], out_hbm.at[pl.ds(base, B2)], semO[b])

        fire_a(0, 0)
        fire_a(1, 1)
        fire_b(0, 0)

        def body(jj, carry):
            j0 = jj * 3
            for t in range(3):
                j = j0 + t

                @pl.when(j + 2 < NB2)
                def _():
                    fire_a(j + 2, (t + 2) % 3)

                @pl.when(j + 1 < NB2)
                def _():
                    fire_b(j + 1, (t + 1) % 3)

                @pl.when(j < NB2)
                def _():
                    work(j, t)

            return carry

        lax.fori_loop(0, (NB2 + 2) // 3, body, 0)
        for j in (NB2 - 3, NB2 - 2, NB2 - 1):
            b = j % 3
            pltpu.make_async_copy(
                s_v.at[b], out_hbm.at[pl.ds(base0 + j * B2, B2)],
                semO[b]).wait()

    return k(Hl, Hr, rev, wa)


EPT3 = E // NS
B3 = 80
NB3 = EPT3 // B3


def _aggregate_call(scores, dest, src, rev, M2):
    @functools.partial(
        pl.kernel,
        out_type=[jax.ShapeDtypeStruct((2 * E, DH), jnp.float32),
                  jax.ShapeDtypeStruct((NC * E,), jnp.float32),
                  jax.ShapeDtypeStruct((NC * N, DH), jnp.float32)],
        mesh=_SC_MESH,
        compiler_params=pltpu.CompilerParams(needs_layout_passes=False),
        scratch_types=[
            pltpu.VMEM((B3,), jnp.int32),
            pltpu.VMEM((2, B3), jnp.float32),
            pltpu.VMEM((2, B3), jnp.int32),
            pltpu.VMEM((2, B3), jnp.float32),
            pltpu.VMEM((2, B3), jnp.float32),
            pltpu.VMEM((2, B3), jnp.int32),
            pltpu.VMEM((2, B3), jnp.int32),
            pltpu.VMEM((2, B3), jnp.int32),
            pltpu.VMEM((2, B3), jnp.int32),
            pltpu.VMEM((2, B3), jnp.int32),
            pltpu.VMEM((2, B3), jnp.float32),
            pltpu.VMEM((2, B3, DH), jnp.float32),
            pltpu.VMEM((2, B3, DH), jnp.float32),
            pltpu.VMEM((40, DH), jnp.float32),
            pltpu.VMEM((1008,), jnp.float32),
            pltpu.VMEM_SHARED((N,), jnp.float32),
            pltpu.VMEM_SHARED((N, DH), jnp.float32),
            pltpu.SemaphoreType.DMA,
            pltpu.SemaphoreType.DMA,
            pltpu.SemaphoreType.DMA,
            pltpu.SemaphoreType.DMA,
            pltpu.SemaphoreType.DMA,
            pltpu.SemaphoreType.DMA,
            pltpu.SemaphoreType.DMA,
            pltpu.SemaphoreType.DMA,
            pltpu.SemaphoreType.DMA,
            pltpu.SemaphoreType.DMA,
            pltpu.SemaphoreType.DMA,
            pltpu.SemaphoreType.DMA,
            pltpu.SemaphoreType.DMA,
        ],
    )
    def k(scores_hbm, dest_hbm, src_hbm, rev_hbm, m2_hbm,
          out_hbm, alpha_hbm, mvh_hbm,
          pat2, tmp_s, dtmp, sg, av, midx, sidx, ridx, ari, wl, arv,
          rowa, rowb, zrow_v, zs_v, s_sh, mv_sh,
          semL0, semL1, semD0, semD1, semG0, semG1, semM0, semM1,
          semW0, semW1, semS, semO0, semO1):
        semL = (semL0, semL1)
        semD = (semD0, semD1)
        semG = (semG0, semG1)
        semM = (semM0, semM1)
        semW = (semW0, semW1)
        semO = (semO0, semO1)
        c = lax.axis_index("c")
        s = lax.axis_index("s")
        chunk0 = s * EPT3
        aoff = c * E + chunk0

        def pipe2(nblk, fire, work):
            fire(0, 0)

            def body(jj, carry):
                j0 = jj * 2

                @pl.when(j0 + 1 < nblk)
                def _():
                    fire(j0 + 1, 1)

                work(j0, 0)

                @pl.when(j0 + 2 < nblk)
                def _():
                    fire(j0 + 2, 0)

                @pl.when(j0 + 1 < nblk)
                def _():
                    work(j0 + 1, 1)

                return carry

            lax.fori_loop(0, (nblk + 1) // 2, body, 0)

        for k16 in range(B3 // L):
            sl = pl.ds(k16 * L, L)
            pat2[sl] = lax.iota(jnp.int32, L) * 2 + 2 * k16 * L

        zero16 = jnp.zeros((L,), jnp.float32)

        def zr_init(r, carry):
            for k16 in range(DH // L):
                zrow_v[r, pl.ds(k16 * L, L)] = zero16
            return carry

        lax.fori_loop(0, 40, zr_init, 0)

        def zs_init(k16, carry):
            zs_v[pl.ds(k16 * L, L)] = zero16
            return carry

        lax.fori_loop(0, 1008 // L, zs_init, 0)

        @pl.when(s < 10)
        def _zero():
            pltpu.sync_copy(zs_v.at[pl.ds(0, 1000)],
                            s_sh.at[pl.ds(s * 1000, 1000)])

            def zmv(t, carry):
                pltpu.sync_copy(zrow_v,
                                mv_sh.at[pl.ds(s * 1000 + t * 40, 40)])
                return carry

            lax.fori_loop(0, 25, zmv, 0)

        plsc.subcore_barrier()

        def fire_a(j, b):
            off = chunk0 + j * B3

            @pl.when(j >= 2)
            def _():
                pltpu.make_async_copy(av.at[b], s_sh.at[dtmp.at[b]],
                                      semG[b]).wait()

            pltpu.async_copy(scores_hbm.at[pl.ds(off, B3)], tmp_s.at[b],
                             semL[b])
            pltpu.async_copy(dest_hbm.at[pl.ds(off, B3)], dtmp.at[b], semD[b])

        def work_a(j, b):
            off = chunk0 + j * B3
            pltpu.make_async_copy(scores_hbm.at[pl.ds(off, B3)], tmp_s.at[b],
                                  semL[b]).wait()
            pltpu.make_async_copy(dest_hbm.at[pl.ds(off, B3)], dtmp.at[b],
                                  semD[b]).wait()
            for k16 in range(B3 // L):
                sl = pl.ds(k16 * L, L)
                av[b, sl] = jnp.exp(tmp_s[b, sl])
            pltpu.async_copy(av.at[b], s_sh.at[dtmp.at[b]], semG[b], add=True)

        pipe2(NB3, fire_a, work_a)
        for j in (NB3 - 2, NB3 - 1):
            b = j % 2
            pltpu.make_async_copy(av.at[b], s_sh.at[dtmp.at[b]],
                                  semG[b]).wait()
        plsc.subcore_barrier()

        def fire_bc(j, b):
            off = chunk0 + j * B3
            off2 = 2 * off + c

            @pl.when(j >= 2)
            def _():
                off_p = c * E + chunk0 + (j - 2) * B3
                pltpu.make_async_copy(
                    av.at[b], alpha_hbm.at[pl.ds(off_p, B3)], semW[b]).wait()
                pltpu.make_async_copy(rowa.at[b], mv_sh.at[dtmp.at[b]],
                                      semG[b]).wait()

            for k16 in range(B3 // L):
                sl = pl.ds(k16 * L, L)
                midx[b, sl] = pat2[sl] + off2
            pltpu.async_copy(m2_hbm.at[midx.at[b]], rowa.at[b], semM[b])
            pltpu.async_copy(scores_hbm.at[pl.ds(off, B3)], tmp_s.at[b],
                             semL[b])
            pltpu.async_copy(dest_hbm.at[pl.ds(off, B3)], dtmp.at[b], semD[b])

        def work_bc(j, b):
            off = chunk0 + j * B3
            pltpu.make_async_copy(scores_hbm.at[pl.ds(off, B3)], tmp_s.at[b],
                                  semL[b]).wait()
            pltpu.make_async_copy(dest_hbm.at[pl.ds(off, B3)], dtmp.at[b],
                                  semD[b]).wait()
            pltpu.async_copy(s_sh.at[dtmp.at[b]], sg.at[b], semS).wait()
            for k16 in range(B3 // L):
                sl = pl.ds(k16 * L, L)
                av[b, sl] = jnp.exp(tmp_s[b, sl]) / (sg[b, sl] + 1e-16)
            pltpu.make_async_copy(m2_hbm.at[midx.at[b]], rowa.at[b],
                                  semM[b]).wait()

            def row(i, c2):
                a16 = plsc.load_gather(av.at[b],
                                       [jnp.full((L,), i, jnp.int32)])
                for k16 in range(DH // L):
                    sl = pl.ds(k16 * L, L)
                    rowa[b, i, sl] = rowa[b, i, sl] * a16
                return c2

            lax.fori_loop(0, B3, row, 0)
            pltpu.async_copy(rowa.at[b], mv_sh.at[dtmp.at[b]], semG[b],
                             add=True)
            pltpu.async_copy(av.at[b],
                             alpha_hbm.at[pl.ds(c * E + off, B3)], semW[b])

        pipe2(NB3, fire_bc, work_bc)
        for j in (NB3 - 2, NB3 - 1):
            b = j % 2
            pltpu.make_async_copy(
                av.at[b], alpha_hbm.at[pl.ds(c * E + chunk0 + j * B3, B3)],
                semW[b]).wait()
            pltpu.make_async_copy(rowa.at[b], mv_sh.at[dtmp.at[b]],
                                  semG[b]).wait()
        plsc.subcore_barrier()

        @pl.when(s < 10)
        def _dump():
            pltpu.sync_copy(mv_sh.at[pl.ds(s * 1000, 1000)],
                            mvh_hbm.at[pl.ds(c * N + s * 1000, 1000)])

        plsc.subcore_barrier()

        def fire_e(j, b):
            off = chunk0 + j * B3

            @pl.when(j >= 2)
            def _():
                pltpu.make_async_copy(rowb.at[b], out_hbm.at[wl.at[b]],
                                      semO[b]).wait()

            pltpu.async_copy(src_hbm.at[pl.ds(off, B3)], sidx.at[b], semL[b])
            pltpu.async_copy(rev_hbm.at[pl.ds(off, B3)], ridx.at[b], semD[b])

        def work_e(j, b):
            off = chunk0 + j * B3
            off2 = 2 * off + c
            pltpu.make_async_copy(src_hbm.at[pl.ds(off, B3)], sidx.at[b],
                                  semL[b]).wait()
            pltpu.make_async_copy(rev_hbm.at[pl.ds(off, B3)], ridx.at[b],
                                  semD[b]).wait()
            for k16 in range(B3 // L):
                sl = pl.ds(k16 * L, L)
                r16 = ridx[b, sl]
                ari[b, sl] = r16 + c * E
                midx[b, sl] = r16 * 2 + c
                wl[b, sl] = pat2[sl] + off2
                sidx[b, sl] = sidx[b, sl] + c * N
            cp_mv = pltpu.async_copy(mvh_hbm.at[sidx.at[b]], rowb.at[b],
                                     semG[b])
            cp_a = pltpu.async_copy(alpha_hbm.at[ari.at[b]], arv.at[b],
                                    semW[b])
            cp_m2 = pltpu.async_copy(m2_hbm.at[midx.at[b]], rowa.at[b],
                                     semM[b])
            cp_mv.wait()
            cp_a.wait()
            cp_m2.wait()

            def row(i, c2):
                a16 = plsc.load_gather(arv.at[b],
                                       [jnp.full((L,), i, jnp.int32)])
                for k16 in range(DH // L):
                    sl = pl.ds(k16 * L, L)
                    rowb[b, i, sl] = rowb[b, i, sl] - a16 * rowa[b, i, sl]
                return c2

            lax.fori_loop(0, B3, row, 0)
            pltpu.async_copy(rowb.at[b], out_hbm.at[wl.at[b]], semO[b])

        pipe2(NB3, fire_e, work_e)
        for j in (NB3 - 2, NB3 - 1):
            b = j % 2
            pltpu.make_async_copy(rowb.at[b], out_hbm.at[wl.at[b]],
                                  semO[b]).wait()

    return k(scores, dest, src, rev, M2)


def kernel(M, edge_index, rev_index, dim_size, Wl, bl, Wr, br, Wa, ba):
    del dim_size, ba
    src = edge_index[0]
    dest = edge_index[1]
    rev = rev_index.astype(jnp.int32)
    Hl, Hr = _linear_parts(M, Wl, bl, Wr, br)
    scores = _scores_call(Hl, Hr, rev, Wa.reshape(D))
    M2 = M.reshape(2 * E, DH)
    out2, _alpha, _mvh = _aggregate_call(scores, dest, src, rev, M2)
    return out2.reshape(E, D)


# speedup vs baseline: 1.1237x; 1.1237x over previous
import functools

import jax
import jax.numpy as jnp
from jax import lax
from jax.experimental import pallas as pl
from jax.experimental.pallas import tpu as pltpu
from jax.experimental.pallas import tpu_sc as plsc

E = 160000
N = 10000
D = 256
DH = D // 2
NC, NS, L = 2, 16, 16
NW = NC * NS

BM = 2000


def _mm_body(m_ref, wl_ref, bl_ref, wr_ref, br_ref, hl_ref, hr_ref):
    x = m_ref[...]
    dn = (((1,), (1,)), ((), ()))
    hl_ref[...] = lax.dot_general(
        x, wl_ref[...], dn, preferred_element_type=jnp.float32,
        precision=lax.Precision.DEFAULT) + bl_ref[...]
    hr_ref[...] = lax.dot_general(
        x, wr_ref[...], dn, preferred_element_type=jnp.float32,
        precision=lax.Precision.DEFAULT) + br_ref[...]


def _linear_parts(M, Wl, bl, Wr, br):
    return pl.pallas_call(
        _mm_body,
        grid=(E // BM,),
        in_specs=[
            pl.BlockSpec((BM, D), lambda i: (i, 0)),
            pl.BlockSpec((D, D), lambda i: (0, 0)),
            pl.BlockSpec((1, D), lambda i: (0, 0)),
            pl.BlockSpec((D, D), lambda i: (0, 0)),
            pl.BlockSpec((1, D), lambda i: (0, 0)),
        ],
        out_specs=[pl.BlockSpec((BM, D), lambda i: (i, 0)),
                   pl.BlockSpec((BM, D), lambda i: (i, 0))],
        out_shape=[jax.ShapeDtypeStruct((E, D), jnp.float32),
                   jax.ShapeDtypeStruct((E, D), jnp.float32)],
    )(M, Wl, bl.reshape(1, D), Wr, br.reshape(1, D))


_SC_MESH = plsc.VectorSubcoreMesh(core_axis_name="c", subcore_axis_name="s")

_GDN = lax.GatherDimensionNumbers(
    offset_dims=(), collapsed_slice_dims=(0,), start_index_map=(0,))


def _shuffle(v, idx16):
    return lax.gather(v, idx16.reshape(L, 1), _GDN, (1,),
                      mode=lax.GatherScatterMode.PROMISE_IN_BOUNDS)


def _hsum(v):
    lanes = lax.iota(jnp.int32, L)
    for sh in (1, 2, 4, 8):
        v = v + _shuffle(v, lanes ^ sh)
    return v


EPT2 = E // NW
B2 = 40
NB2 = EPT2 // B2


def _scores_call(Hl, Hr, rev, wa):
    @functools.partial(
        pl.kernel,
        out_type=jax.ShapeDtypeStruct((E,), jnp.float32),
        mesh=_SC_MESH,
        compiler_params=pltpu.CompilerParams(needs_layout_passes=False),
        scratch_types=[
            pltpu.VMEM((D,), jnp.float32),
            pltpu.VMEM((3, B2), jnp.int32),
            pltpu.VMEM((3, B2, D), jnp.float32),
            pltpu.VMEM((3, B2, D), jnp.float32),
            pltpu.VMEM((3, B2), jnp.float32),
            pltpu.SemaphoreType.DMA,
            pltpu.SemaphoreType.DMA,
            pltpu.SemaphoreType.DMA,
            pltpu.SemaphoreType.DMA,
            pltpu.SemaphoreType.DMA,
            pltpu.SemaphoreType.DMA,
            pltpu.SemaphoreType.DMA,
            pltpu.SemaphoreType.DMA,
            pltpu.SemaphoreType.DMA,
            pltpu.SemaphoreType.DMA,
            pltpu.SemaphoreType.DMA,
            pltpu.SemaphoreType.DMA,
        ],
    )
    def k(hl_hbm, hr_hbm, rev_hbm, wa_hbm, out_hbm,
          wa_v, rev_v, hl_v, hr_v, s_v,
          semR0, semR1, semR2, semH0, semH1, semH2,
          semG0, semG1, semG2, semO0, semO1, semO2):
        semR = (semR0, semR1, semR2)
        semH = (semH0, semH1, semH2)
        semG = (semG0, semG1, semG2)
        semO = (semO0, semO1, semO2)
        wid = lax.axis_index("s") * NC + lax.axis_index("c")
        base0 = wid * EPT2
        pltpu.sync_copy(wa_hbm, wa_v)
        lane0 = lax.iota(jnp.int32, L) == 0

        def fire_a(j, b):
            base = base0 + j * B2
            @pl.when(j >= 3)
            def _():
                pltpu.make_async_copy(
                    s_v.at[b], out_hbm.at[pl.ds(base0 + (j - 3) * B2, B2)],
                    semO[b]).wait()

            pltpu.async_copy(rev_hbm.at[pl.ds(base, B2)], rev_v.at[b], semR[b])
            pltpu.async_copy(hl_hbm.at[pl.ds(base, B2)], hl_v.at[b], semH[b])

        def fire_b(j, b):
            base = base0 + j * B2
            pltpu.make_async_copy(rev_hbm.at[pl.ds(base, B2)], rev_v.at[b],
                                  semR[b]).wait()
            pltpu.async_copy(hr_hbm.at[rev_v.at[b]], hr_v.at[b], semG[b])

        def work(j, b):
            base = base0 + j * B2
            pltpu.make_async_copy(hl_hbm.at[pl.ds(base, B2)], hl_v.at[b],
                                  semH[b]).wait()
            pltpu.make_async_copy(hr_hbm.at[rev_v.at[b]], hr_v.at[b],
                                  semG[b]).wait()

            def edge(i, c2):
                acc = jnp.zeros((L,), jnp.float32)
                for k16 in range(D // L):
                    sl = pl.ds(k16 * L, L)
                    t = hl_v[b, i, sl] + hr_v[b, i, sl]
                    t = jnp.where(t >= 0.0, t, 0.2 * t)
                    acc = acc + t * wa_v[sl]
                plsc.store_scatter(s_v.at[b], [jnp.full((L,), i, jnp.int32)],
                                   _hsum(acc), mask=lane0)
                return c2

            lax.fori_loop(0, B2, edge, 0)
            pltpu.async_copy(s_v.at[b], out_hbm.at[pl.ds(base, B2)], semO[b])

        fire_a(0, 0)
        fire_a(1, 1)
        fire_b(0, 0)

        def body(jj, carry):
            j0 = jj * 3
            for t in range(3):
                j = j0 + t

                @pl.when(j + 2 < NB2)
                def _():
                    fire_a(j + 2, (t + 2) % 3)

                @pl.when(j + 1 < NB2)
                def _():
                    fire_b(j + 1, (t + 1) % 3)

                @pl.when(j < NB2)
                def _():
                    work(j, t)

            return carry

        lax.fori_loop(0, (NB2 + 2) // 3, body, 0)
        for j in (NB2 - 3, NB2 - 2, NB2 - 1):
            b = j % 3
            pltpu.make_async_copy(
                s_v.at[b], out_hbm.at[pl.ds(base0 + j * B2, B2)],
                semO[b]).wait()

    return k(Hl, Hr, rev, wa)


EPT3 = E // NS
B3 = 80
NB3 = EPT3 // B3


def _aggregate_call(scores, dest, src, rev, M2):
    @functools.partial(
        pl.kernel,
        out_type=[jax.ShapeDtypeStruct((2 * E, DH), jnp.float32),
                  jax.ShapeDtypeStruct((NC * E,), jnp.float32)],
        mesh=_SC_MESH,
        compiler_params=pltpu.CompilerParams(needs_layout_passes=False),
        scratch_types=[
            pltpu.VMEM((B3,), jnp.int32),
            pltpu.VMEM((2, B3), jnp.float32),
            pltpu.VMEM((2, B3), jnp.int32),
            pltpu.VMEM((2, B3), jnp.float32),
            pltpu.VMEM((2, B3), jnp.float32),
            pltpu.VMEM((2, B3), jnp.int32),
            pltpu.VMEM((2, B3), jnp.int32),
            pltpu.VMEM((2, B3), jnp.int32),
            pltpu.VMEM((2, B3), jnp.int32),
            pltpu.VMEM((2, B3), jnp.int32),
            pltpu.VMEM((2, B3), jnp.float32),
            pltpu.VMEM((2, B3, DH), jnp.float32),
            pltpu.VMEM((2, B3, DH), jnp.float32),
            pltpu.VMEM((40, DH), jnp.float32),
            pltpu.VMEM((1008,), jnp.float32),
            pltpu.VMEM_SHARED((N,), jnp.float32),
            pltpu.VMEM_SHARED((N, DH), jnp.float32),
            pltpu.SemaphoreType.DMA,
            pltpu.SemaphoreType.DMA,
            pltpu.SemaphoreType.DMA,
            pltpu.SemaphoreType.DMA,
            pltpu.SemaphoreType.DMA,
            pltpu.SemaphoreType.DMA,
            pltpu.SemaphoreType.DMA,
            pltpu.SemaphoreType.DMA,
            pltpu.SemaphoreType.DMA,
            pltpu.SemaphoreType.DMA,
            pltpu.SemaphoreType.DMA,
            pltpu.SemaphoreType.DMA,
            pltpu.SemaphoreType.DMA,
        ],
    )
    def k(scores_hbm, dest_hbm, src_hbm, rev_hbm, m2_hbm,
          out_hbm, alpha_hbm,
          pat2, tmp_s, dtmp, sg, av, midx, sidx, ridx, ari, wl, arv,
          rowa, rowb, zrow_v, zs_v, s_sh, mv_sh,
          semL0, semL1, semD0, semD1, semG0, semG1, semM0, semM1,
          semW0, semW1, semS, semO0, semO1):
        semL = (semL0, semL1)
        semD = (semD0, semD1)
        semG = (semG0, semG1)
        semM = (semM0, semM1)
        semW = (semW0, semW1)
        semO = (semO0, semO1)
        c = lax.axis_index("c")
        s = lax.axis_index("s")
        chunk0 = s * EPT3
        aoff = c * E + chunk0

        def pipe2(nblk, fire, work):
            fire(0, 0)

            def body(jj, carry):
                j0 = jj * 2

                @pl.when(j0 + 1 < nblk)
                def _():
                    fire(j0 + 1, 1)

                work(j0, 0)

                @pl.when(j0 + 2 < nblk)
                def _():
                    fire(j0 + 2, 0)

                @pl.when(j0 + 1 < nblk)
                def _():
                    work(j0 + 1, 1)

                return carry

            lax.fori_loop(0, (nblk + 1) // 2, body, 0)

        for k16 in range(B3 // L):
            sl = pl.ds(k16 * L, L)
            pat2[sl] = lax.iota(jnp.int32, L) * 2 + 2 * k16 * L

        zero16 = jnp.zeros((L,), jnp.float32)

        def zr_init(r, carry):
            for k16 in range(DH // L):
                zrow_v[r, pl.ds(k16 * L, L)] = zero16
            return carry

        lax.fori_loop(0, 40, zr_init, 0)

        def zs_init(k16, carry):
            zs_v[pl.ds(k16 * L, L)] = zero16
            return carry

        lax.fori_loop(0, 1008 // L, zs_init, 0)

        @pl.when(s < 10)
        def _zero():
            pltpu.sync_copy(zs_v.at[pl.ds(0, 1000)],
                            s_sh.at[pl.ds(s * 1000, 1000)])

            def zmv(t, carry):
                pltpu.sync_copy(zrow_v,
                                mv_sh.at[pl.ds(s * 1000 + t * 40, 40)])
                return carry

            lax.fori_loop(0, 25, zmv, 0)

        plsc.subcore_barrier()

        def fire_a(j, b):
            off = chunk0 + j * B3

            @pl.when(j >= 2)
            def _():
                pltpu.make_async_copy(av.at[b], s_sh.at[dtmp.at[b]],
                                      semG[b]).wait()

            pltpu.async_copy(scores_hbm.at[pl.ds(off, B3)], tmp_s.at[b],
                             semL[b])
            pltpu.async_copy(dest_hbm.at[pl.ds(off, B3)], dtmp.at[b], semD[b])

        def work_a(j, b):
            off = chunk0 + j * B3
            pltpu.make_async_copy(scores_hbm.at[pl.ds(off, B3)], tmp_s.at[b],
                                  semL[b]).wait()
            pltpu.make_async_copy(dest_hbm.at[pl.ds(off, B3)], dtmp.at[b],
                                  semD[b]).wait()
            for k16 in range(B3 // L):
                sl = pl.ds(k16 * L, L)
                av[b, sl] = jnp.exp(tmp_s[b, sl])
            pltpu.async_copy(av.at[b], s_sh.at[dtmp.at[b]], semG[b], add=True)

        pipe2(NB3, fire_a, work_a)
        for j in (NB3 - 2, NB3 - 1):
            b = j % 2
            pltpu.make_async_copy(av.at[b], s_sh.at[dtmp.at[b]],
                                  semG[b]).wait()
        plsc.subcore_barrier()

        def fire_bc(j, b):
            off = chunk0 + j * B3
            off2 = 2 * off + c

            @pl.when(j >= 2)
            def _():
                off_p = c * E + chunk0 + (j - 2) * B3
                pltpu.make_async_copy(
                    av.at[b], alpha_hbm.at[pl.ds(off_p, B3)], semW[b]).wait()
                pltpu.make_async_copy(rowa.at[b], mv_sh.at[dtmp.at[b]],
                                      semG[b]).wait()

            for k16 in range(B3 // L):
                sl = pl.ds(k16 * L, L)
                midx[b, sl] = pat2[sl] + off2
            pltpu.async_copy(m2_hbm.at[midx.at[b]], rowa.at[b], semM[b])
            pltpu.async_copy(scores_hbm.at[pl.ds(off, B3)], tmp_s.at[b],
                             semL[b])
            pltpu.async_copy(dest_hbm.at[pl.ds(off, B3)], dtmp.at[b], semD[b])

        def work_bc(j, b):
            off = chunk0 + j * B3
            pltpu.make_async_copy(scores_hbm.at[pl.ds(off, B3)], tmp_s.at[b],
                                  semL[b]).wait()
            pltpu.make_async_copy(dest_hbm.at[pl.ds(off, B3)], dtmp.at[b],
                                  semD[b]).wait()
            pltpu.async_copy(s_sh.at[dtmp.at[b]], sg.at[b], semS).wait()
            for k16 in range(B3 // L):
                sl = pl.ds(k16 * L, L)
                av[b, sl] = jnp.exp(tmp_s[b, sl]) / (sg[b, sl] + 1e-16)
            pltpu.make_async_copy(m2_hbm.at[midx.at[b]], rowa.at[b],
                                  semM[b]).wait()

            def row(i2, c2):
                for u in range(2):
                    i = i2 * 2 + u
                    a16 = plsc.load_gather(av.at[b],
                                           [jnp.full((L,), i, jnp.int32)])
                    for k16 in range(DH // L):
                        sl = pl.ds(k16 * L, L)
                        rowa[b, i, sl] = rowa[b, i, sl] * a16
                return c2

            lax.fori_loop(0, B3 // 2, row, 0)
            pltpu.async_copy(rowa.at[b], mv_sh.at[dtmp.at[b]], semG[b],
                             add=True)
            pltpu.async_copy(av.at[b],
                             alpha_hbm.at[pl.ds(c * E + off, B3)], semW[b])

        pipe2(NB3, fire_bc, work_bc)
        for j in (NB3 - 2, NB3 - 1):
            b = j % 2
            pltpu.make_async_copy(
                av.at[b], alpha_hbm.at[pl.ds(c * E + chunk0 + j * B3, B3)],
                semW[b]).wait()
            pltpu.make_async_copy(rowa.at[b], mv_sh.at[dtmp.at[b]],
                                  semG[b]).wait()
        plsc.subcore_barrier()

        def fire_e(j, b):
            off = chunk0 + j * B3

            @pl.when(j >= 2)
            def _():
                pltpu.make_async_copy(rowb.at[b], out_hbm.at[wl.at[b]],
                                      semO[b]).wait()

            pltpu.async_copy(src_hbm.at[pl.ds(off, B3)], sidx.at[b], semL[b])
            pltpu.async_copy(rev_hbm.at[pl.ds(off, B3)], ridx.at[b], semD[b])

        def work_e(j, b):
            off = chunk0 + j * B3
            off2 = 2 * off + c
            pltpu.make_async_copy(src_hbm.at[pl.ds(off, B3)], sidx.at[b],
                                  semL[b]).wait()
            pltpu.make_async_copy(rev_hbm.at[pl.ds(off, B3)], ridx.at[b],
                                  semD[b]).wait()
            for k16 in range(B3 // L):
                sl = pl.ds(k16 * L, L)
                r16 = ridx[b, sl]
                ari[b, sl] = r16 + c * E
                midx[b, sl] = r16 * 2 + c
                wl[b, sl] = pat2[sl] + off2
            cp_mv = pltpu.async_copy(mv_sh.at[sidx.at[b]], rowb.at[b],
                                     semG[b])
            cp_a = pltpu.async_copy(alpha_hbm.at[ari.at[b]], arv.at[b],
                                    semW[b])
            cp_m2 = pltpu.async_copy(m2_hbm.at[midx.at[b]], rowa.at[b],
                                     semM[b])
            cp_mv.wait()
            cp_a.wait()
            cp_m2.wait()

            def row(i2, c2):
                for u in range(2):
                    i = i2 * 2 + u
                    a16 = plsc.load_gather(arv.at[b],
                                           [jnp.full((L,), i, jnp.int32)])
                    for k16 in range(DH // L):
                        sl = pl.ds(k16 * L, L)
                        rowb[b, i, sl] = rowb[b, i, sl] - a16 * rowa[b, i, sl]
                return c2

            lax.fori_loop(0, B3 // 2, row, 0)
            pltpu.async_copy(rowb.at[b], out_hbm.at[wl.at[b]], semO[b])

        pipe2(NB3, fire_e, work_e)
        for j in (NB3 - 2, NB3 - 1):
            b = j % 2
            pltpu.make_async_copy(rowb.at[b], out_hbm.at[wl.at[b]],
                                  semO[b]).wait()

    return k(scores, dest, src, rev, M2)


def kernel(M, edge_index, rev_index, dim_size, Wl, bl, Wr, br, Wa, ba):
    del dim_size, ba
    src = edge_index[0]
    dest = edge_index[1]
    rev = rev_index.astype(jnp.int32)
    Hl, Hr = _linear_parts(M, Wl, bl, Wr, br)
    scores = _scores_call(Hl, Hr, rev, Wa.reshape(D))
    M2 = M.reshape(2 * E, DH)
    out2, _alpha = _aggregate_call(scores, dest, src, rev, M2)
    return out2.reshape(E, D)


# speedup vs baseline: 1.1365x; 1.0114x over previous
import functools

import jax
import jax.numpy as jnp
from jax import lax
from jax.experimental import pallas as pl
from jax.experimental.pallas import tpu as pltpu
from jax.experimental.pallas import tpu_sc as plsc

E = 160000
N = 10000
D = 256
DH = D // 2
NC, NS, L = 2, 16, 16
NW = NC * NS

BM = 2000


def _mm_body(m_ref, wl_ref, bl_ref, wr_ref, br_ref, hl_ref, hr_ref):
    x = m_ref[...]
    dn = (((1,), (1,)), ((), ()))
    hl_ref[...] = lax.dot_general(
        x, wl_ref[...], dn, preferred_element_type=jnp.float32,
        precision=lax.Precision.DEFAULT) + bl_ref[...]
    hr_ref[...] = lax.dot_general(
        x, wr_ref[...], dn, preferred_element_type=jnp.float32,
        precision=lax.Precision.DEFAULT) + br_ref[...]


def _linear_parts(M, Wl, bl, Wr, br):
    return pl.pallas_call(
        _mm_body,
        grid=(E // BM,),
        in_specs=[
            pl.BlockSpec((BM, D), lambda i: (i, 0)),
            pl.BlockSpec((D, D), lambda i: (0, 0)),
            pl.BlockSpec((1, D), lambda i: (0, 0)),
            pl.BlockSpec((D, D), lambda i: (0, 0)),
            pl.BlockSpec((1, D), lambda i: (0, 0)),
        ],
        out_specs=[pl.BlockSpec((BM, D), lambda i: (i, 0)),
                   pl.BlockSpec((BM, D), lambda i: (i, 0))],
        out_shape=[jax.ShapeDtypeStruct((E, D), jnp.float32),
                   jax.ShapeDtypeStruct((E, D), jnp.float32)],
    )(M, Wl, bl.reshape(1, D), Wr, br.reshape(1, D))


_SC_MESH = plsc.VectorSubcoreMesh(core_axis_name="c", subcore_axis_name="s")

_GDN = lax.GatherDimensionNumbers(
    offset_dims=(), collapsed_slice_dims=(0,), start_index_map=(0,))


def _shuffle(v, idx16):
    return lax.gather(v, idx16.reshape(L, 1), _GDN, (1,),
                      mode=lax.GatherScatterMode.PROMISE_IN_BOUNDS)


def _hsum(v):
    lanes = lax.iota(jnp.int32, L)
    for sh in (1, 2, 4, 8):
        v = v + _shuffle(v, lanes ^ sh)
    return v


EPT2 = E // NW
B2 = 40
NB2 = EPT2 // B2


def _scores_call(Hl, Hr, rev, wa):
    @functools.partial(
        pl.kernel,
        out_type=jax.ShapeDtypeStruct((E,), jnp.float32),
        mesh=_SC_MESH,
        compiler_params=pltpu.CompilerParams(needs_layout_passes=False),
        scratch_types=[
            pltpu.VMEM((D,), jnp.float32),
            pltpu.VMEM((3, B2), jnp.int32),
            pltpu.VMEM((3, B2, D), jnp.float32),
            pltpu.VMEM((3, B2, D), jnp.float32),
            pltpu.VMEM((3, B2), jnp.float32),
            pltpu.SemaphoreType.DMA,
            pltpu.SemaphoreType.DMA,
            pltpu.SemaphoreType.DMA,
            pltpu.SemaphoreType.DMA,
            pltpu.SemaphoreType.DMA,
            pltpu.SemaphoreType.DMA,
            pltpu.SemaphoreType.DMA,
            pltpu.SemaphoreType.DMA,
            pltpu.SemaphoreType.DMA,
            pltpu.SemaphoreType.DMA,
            pltpu.SemaphoreType.DMA,
            pltpu.SemaphoreType.DMA,
        ],
    )
    def k(hl_hbm, hr_hbm, rev_hbm, wa_hbm, out_hbm,
          wa_v, rev_v, hl_v, hr_v, s_v,
          semR0, semR1, semR2, semH0, semH1, semH2,
          semG0, semG1, semG2, semO0, semO1, semO2):
        semR = (semR0, semR1, semR2)
        semH = (semH0, semH1, semH2)
        semG = (semG0, semG1, semG2)
        semO = (semO0, semO1, semO2)
        wid = lax.axis_index("s") * NC + lax.axis_index("c")
        base0 = wid * EPT2
        pltpu.sync_copy(wa_hbm, wa_v)
        lane0 = lax.iota(jnp.int32, L) == 0
        wa_regs = [wa_v[pl.ds(k16 * L, L)] for k16 in range(D // L)]

        def fire_a(j, b):
            base = base0 + j * B2
            @pl.when(j >= 3)
            def _():
                pltpu.make_async_copy(
                    s_v.at[b], out_hbm.at[pl.ds(base0 + (j - 3) * B2, B2)],
                    semO[b]).wait()

            pltpu.async_copy(rev_hbm.at[pl.ds(base, B2)], rev_v.at[b], semR[b])
            pltpu.async_copy(hl_hbm.at[pl.ds(base, B2)], hl_v.at[b], semH[b])

        def fire_b(j, b):
            base = base0 + j * B2
            pltpu.make_async_copy(rev_hbm.at[pl.ds(base, B2)], rev_v.at[b],
                                  semR[b]).wait()
            pltpu.async_copy(hr_hbm.at[rev_v.at[b]], hr_v.at[b], semG[b])

        def work(j, b):
            base = base0 + j * B2
            pltpu.make_async_copy(hl_hbm.at[pl.ds(base, B2)], hl_v.at[b],
                                  semH[b]).wait()
            pltpu.make_async_copy(hr_hbm.at[rev_v.at[b]], hr_v.at[b],
                                  semG[b]).wait()

            def edge(i, c2):
                acc = jnp.zeros((L,), jnp.float32)
                for k16 in range(D // L):
                    sl = pl.ds(k16 * L, L)
                    t = hl_v[b, i, sl] + hr_v[b, i, sl]
                    t = jnp.where(t >= 0.0, t, 0.2 * t)
                    acc = acc + t * wa_regs[k16]
                plsc.store_scatter(s_v.at[b], [jnp.full((L,), i, jnp.int32)],
                                   _hsum(acc), mask=lane0)
                return c2

            lax.fori_loop(0, B2, edge, 0)
            pltpu.async_copy(s_v.at[b], out_hbm.at[pl.ds(base, B2)], semO[b])

        fire_a(0, 0)
        fire_a(1, 1)
        fire_b(0, 0)

        def body(jj, carry):
            j0 = jj * 3
            for t in range(3):
                j = j0 + t

                @pl.when(j + 2 < NB2)
                def _():
                    fire_a(j + 2, (t + 2) % 3)

                @pl.when(j + 1 < NB2)
                def _():
                    fire_b(j + 1, (t + 1) % 3)

                @pl.when(j < NB2)
                def _():
                    work(j, t)

            return carry

        lax.fori_loop(0, (NB2 + 2) // 3, body, 0)
        for j in (NB2 - 3, NB2 - 2, NB2 - 1):
            b = j % 3
            pltpu.make_async_copy(
                s_v.at[b], out_hbm.at[pl.ds(base0 + j * B2, B2)],
                semO[b]).wait()

    return k(Hl, Hr, rev, wa)


EPT3 = E // NS
B3 = 80
NB3 = EPT3 // B3


def _aggregate_call(scores, dest, src, rev, M2):
    @functools.partial(
        pl.kernel,
        out_type=[jax.ShapeDtypeStruct((2 * E, DH), jnp.float32),
                  jax.ShapeDtypeStruct((NC * E,), jnp.float32)],
        mesh=_SC_MESH,
        compiler_params=pltpu.CompilerParams(needs_layout_passes=False),
        scratch_types=[
            pltpu.VMEM((B3,), jnp.int32),
            pltpu.VMEM((2, B3), jnp.float32),
            pltpu.VMEM((2, B3), jnp.int32),
            pltpu.VMEM((2, B3), jnp.float32),
            pltpu.VMEM((2, B3), jnp.float32),
            pltpu.VMEM((2, B3), jnp.int32),
            pltpu.VMEM((2, B3), jnp.int32),
            pltpu.VMEM((2, B3), jnp.int32),
            pltpu.VMEM((2, B3), jnp.int32),
            pltpu.VMEM((2, B3), jnp.int32),
            pltpu.VMEM((2, B3), jnp.float32),
            pltpu.VMEM((2, B3, DH), jnp.float32),
            pltpu.VMEM((2, B3, DH), jnp.float32),
            pltpu.VMEM((40, DH), jnp.float32),
            pltpu.VMEM((1008,), jnp.float32),
            pltpu.VMEM_SHARED((N,), jnp.float32),
            pltpu.VMEM_SHARED((N, DH), jnp.float32),
            pltpu.SemaphoreType.DMA,
            pltpu.SemaphoreType.DMA,
            pltpu.SemaphoreType.DMA,
            pltpu.SemaphoreType.DMA,
            pltpu.SemaphoreType.DMA,
            pltpu.SemaphoreType.DMA,
            pltpu.SemaphoreType.DMA,
            pltpu.SemaphoreType.DMA,
            pltpu.SemaphoreType.DMA,
            pltpu.SemaphoreType.DMA,
            pltpu.SemaphoreType.DMA,
            pltpu.SemaphoreType.DMA,
            pltpu.SemaphoreType.DMA,
        ],
    )
    def k(scores_hbm, dest_hbm, src_hbm, rev_hbm, m2_hbm,
          out_hbm, alpha_hbm,
          pat2, tmp_s, dtmp, sg, av, midx, sidx, ridx, ari, wl, arv,
          rowa, rowb, zrow_v, zs_v, s_sh, mv_sh,
          semL0, semL1, semD0, semD1, semG0, semG1, semM0, semM1,
          semW0, semW1, semS, semO0, semO1):
        semL = (semL0, semL1)
        semD = (semD0, semD1)
        semG = (semG0, semG1)
        semM = (semM0, semM1)
        semW = (semW0, semW1)
        semO = (semO0, semO1)
        c = lax.axis_index("c")
        s = lax.axis_index("s")
        chunk0 = s * EPT3
        aoff = c * E + chunk0

        def pipe2(nblk, fire, work):
            fire(0, 0)

            def body(jj, carry):
                j0 = jj * 2

                @pl.when(j0 + 1 < nblk)
                def _():
                    fire(j0 + 1, 1)

                work(j0, 0)

                @pl.when(j0 + 2 < nblk)
                def _():
                    fire(j0 + 2, 0)

                @pl.when(j0 + 1 < nblk)
                def _():
                    work(j0 + 1, 1)

                return carry

            lax.fori_loop(0, (nblk + 1) // 2, body, 0)

        for k16 in range(B3 // L):
            sl = pl.ds(k16 * L, L)
            pat2[sl] = lax.iota(jnp.int32, L) * 2 + 2 * k16 * L

        zero16 = jnp.zeros((L,), jnp.float32)

        def zr_init(r, carry):
            for k16 in range(DH // L):
                zrow_v[r, pl.ds(k16 * L, L)] = zero16
            return carry

        lax.fori_loop(0, 40, zr_init, 0)

        def zs_init(k16, carry):
            zs_v[pl.ds(k16 * L, L)] = zero16
            return carry

        lax.fori_loop(0, 1008 // L, zs_init, 0)

        @pl.when(s < 10)
        def _zero():
            pltpu.sync_copy(zs_v.at[pl.ds(0, 1000)],
                            s_sh.at[pl.ds(s * 1000, 1000)])

            def zmv(t, carry):
                pltpu.sync_copy(zrow_v,
                                mv_sh.at[pl.ds(s * 1000 + t * 40, 40)])
                return carry

            lax.fori_loop(0, 25, zmv, 0)

        plsc.subcore_barrier()

        def fire_a(j, b):
            off = chunk0 + j * B3

            @pl.when(j >= 2)
            def _():
                pltpu.make_async_copy(av.at[b], s_sh.at[dtmp.at[b]],
                                      semG[b]).wait()

            pltpu.async_copy(scores_hbm.at[pl.ds(off, B3)], tmp_s.at[b],
                             semL[b])
            pltpu.async_copy(dest_hbm.at[pl.ds(off, B3)], dtmp.at[b], semD[b])

        def work_a(j, b):
            off = chunk0 + j * B3
            pltpu.make_async_copy(scores_hbm.at[pl.ds(off, B3)], tmp_s.at[b],
                                  semL[b]).wait()
            pltpu.make_async_copy(dest_hbm.at[pl.ds(off, B3)], dtmp.at[b],
                                  semD[b]).wait()
            for k16 in range(B3 // L):
                sl = pl.ds(k16 * L, L)
                av[b, sl] = jnp.exp(tmp_s[b, sl])
            pltpu.async_copy(av.at[b], s_sh.at[dtmp.at[b]], semG[b], add=True)

        pipe2(NB3, fire_a, work_a)
        for j in (NB3 - 2, NB3 - 1):
            b = j % 2
            pltpu.make_async_copy(av.at[b], s_sh.at[dtmp.at[b]],
                                  semG[b]).wait()
        plsc.subcore_barrier()

        def fire_bc(j, b):
            off = chunk0 + j * B3
            off2 = 2 * off + c

            @pl.when(j >= 2)
            def _():
                off_p = c * E + chunk0 + (j - 2) * B3
                pltpu.make_async_copy(
                    av.at[b], alpha_hbm.at[pl.ds(off_p, B3)], semW[b]).wait()
                pltpu.make_async_copy(rowa.at[b], mv_sh.at[dtmp.at[b]],
                                      semG[b]).wait()

            for k16 in range(B3 // L):
                sl = pl.ds(k16 * L, L)
                midx[b, sl] = pat2[sl] + off2
            pltpu.async_copy(m2_hbm.at[midx.at[b]], rowa.at[b], semM[b])
            pltpu.async_copy(scores_hbm.at[pl.ds(off, B3)], tmp_s.at[b],
                             semL[b])
            pltpu.async_copy(dest_hbm.at[pl.ds(off, B3)], dtmp.at[b], semD[b])

        def work_bc(j, b):
            off = chunk0 + j * B3
            pltpu.make_async_copy(scores_hbm.at[pl.ds(off, B3)], tmp_s.at[b],
                                  semL[b]).wait()
            pltpu.make_async_copy(dest_hbm.at[pl.ds(off, B3)], dtmp.at[b],
                                  semD[b]).wait()
            pltpu.async_copy(s_sh.at[dtmp.at[b]], sg.at[b], semS).wait()
            for k16 in range(B3 // L):
                sl = pl.ds(k16 * L, L)
                av[b, sl] = jnp.exp(tmp_s[b, sl]) / (sg[b, sl] + 1e-16)
            pltpu.make_async_copy(m2_hbm.at[midx.at[b]], rowa.at[b],
                                  semM[b]).wait()

            def row(i2, c2):
                for u in range(2):
                    i = i2 * 2 + u
                    a16 = plsc.load_gather(av.at[b],
                                           [jnp.full((L,), i, jnp.int32)])
                    for k16 in range(DH // L):
                        sl = pl.ds(k16 * L, L)
                        rowa[b, i, sl] = rowa[b, i, sl] * a16
                return c2

            lax.fori_loop(0, B3 // 2, row, 0)
            pltpu.async_copy(rowa.at[b], mv_sh.at[dtmp.at[b]], semG[b],
                             add=True)
            pltpu.async_copy(av.at[b],
                             alpha_hbm.at[pl.ds(c * E + off, B3)], semW[b])

        pipe2(NB3, fire_bc, work_bc)
        for j in (NB3 - 2, NB3 - 1):
            b = j % 2
            pltpu.make_async_copy(
                av.at[b], alpha_hbm.at[pl.ds(c * E + chunk0 + j * B3, B3)],
                semW[b]).wait()
            pltpu.make_async_copy(rowa.at[b], mv_sh.at[dtmp.at[b]],
                                  semG[b]).wait()
        plsc.subcore_barrier()

        def fire_e(j, b):
            off = chunk0 + j * B3

            @pl.when(j >= 2)
            def _():
                pltpu.make_async_copy(rowb.at[b], out_hbm.at[wl.at[b]],
                                      semO[b]).wait()

            pltpu.async_copy(src_hbm.at[pl.ds(off, B3)], sidx.at[b], semL[b])
            pltpu.async_copy(rev_hbm.at[pl.ds(off, B3)], ridx.at[b], semD[b])

        def work_e(j, b):
            off = chunk0 + j * B3
            off2 = 2 * off + c
            pltpu.make_async_copy(src_hbm.at[pl.ds(off, B3)], sidx.at[b],
                                  semL[b]).wait()
            pltpu.make_async_copy(rev_hbm.at[pl.ds(off, B3)], ridx.at[b],
                                  semD[b]).wait()
            for k16 in range(B3 // L):
                sl = pl.ds(k16 * L, L)
                r16 = ridx[b, sl]
                ari[b, sl] = r16 + c * E
                midx[b, sl] = r16 * 2 + c
                wl[b, sl] = pat2[sl] + off2
            cp_mv = pltpu.async_copy(mv_sh.at[sidx.at[b]], rowb.at[b],
                                     semG[b])
            cp_a = pltpu.async_copy(alpha_hbm.at[ari.at[b]], arv.at[b],
                                    semW[b])
            cp_m2 = pltpu.async_copy(m2_hbm.at[midx.at[b]], rowa.at[b],
                                     semM[b])
            cp_mv.wait()
            cp_a.wait()
            cp_m2.wait()

            def row(i2, c2):
                for u in range(2):
                    i = i2 * 2 + u
                    a16 = plsc.load_gather(arv.at[b],
                                           [jnp.full((L,), i, jnp.int32)])
                    for k16 in range(DH // L):
                        sl = pl.ds(k16 * L, L)
                        rowb[b, i, sl] = rowb[b, i, sl] - a16 * rowa[b, i, sl]
                return c2

            lax.fori_loop(0, B3 // 2, row, 0)
            pltpu.async_copy(rowb.at[b], out_hbm.at[wl.at[b]], semO[b])

        pipe2(NB3, fire_e, work_e)
        for j in (NB3 - 2, NB3 - 1):
            b = j % 2
            pltpu.make_async_copy(rowb.at[b], out_hbm.at[wl.at[b]],
                                  semO[b]).wait()

    return k(scores, dest, src, rev, M2)


def kernel(M, edge_index, rev_index, dim_size, Wl, bl, Wr, br, Wa, ba):
    del dim_size, ba
    src = edge_index[0]
    dest = edge_index[1]
    rev = rev_index.astype(jnp.int32)
    Hl, Hr = _linear_parts(M, Wl, bl, Wr, br)
    scores = _scores_call(Hl, Hr, rev, Wa.reshape(D))
    M2 = M.reshape(2 * E, DH)
    out2, _alpha = _aggregate_call(scores, dest, src, rev, M2)
    return out2.reshape(E, D)
